# baseline (speedup 1.0000x reference)
import functools

import jax
import jax.numpy as jnp
from jax import lax
from jax.experimental import pallas as pl
from jax.experimental.pallas import tpu as pltpu
from jax.experimental.pallas import tpu_sc as plsc

BATCH = 4096
HIST = 50
E_DIMS = 64
SPLITS = 2
SBATCH = BATCH // SPLITS
CHUNK_B = 2
CHUNK_I = CHUNK_B * HIST
CHUNK_IP = 104

_info = plsc.get_sparse_core_info()
_NC, _NS = _info.num_cores, _info.num_subcores
_NW = _NC * _NS
_ROWS_W = SBATCH // _NW
_CHUNKS_W = _ROWS_W // CHUNK_B

_mesh = plsc.VectorSubcoreMesh(core_axis_name="c", subcore_axis_name="s")


@functools.partial(
    pl.kernel,
    mesh=_mesh,
    out_type=jax.ShapeDtypeStruct((SBATCH, HIST, E_DIMS), jnp.float32),
    scratch_types=[
        pltpu.VMEM((_CHUNKS_W, CHUNK_IP), jnp.int32),
        pltpu.VMEM((CHUNK_IP, E_DIMS), jnp.float32),
        pltpu.VMEM((CHUNK_IP, E_DIMS), jnp.float32),
        pltpu.SemaphoreType.DMA,
        pltpu.SemaphoreType.DMA,
    ],
    compiler_params=pltpu.CompilerParams(use_tc_tiling_on_sc=False),
)
def _gather_kernel(tab_hbm, xp_hbm, out_hbm, idx_v, buf0, buf1, sem0, sem1):
    wid = lax.axis_index("s") * _NC + lax.axis_index("c")
    b0 = wid * _ROWS_W
    pltpu.sync_copy(xp_hbm.at[wid], idx_v)

    pltpu.make_async_copy(tab_hbm.at[idx_v.at[0]], buf0, sem0).start()

    def write_out(buf, b):
        pltpu.sync_copy(buf.at[pl.ds(0, HIST), :], out_hbm.at[b])
        pltpu.sync_copy(buf.at[pl.ds(HIST, HIST), :], out_hbm.at[b + 1])

    def body(i, carry):
        c0 = 2 * i
        pltpu.make_async_copy(tab_hbm.at[idx_v.at[c0 + 1]], buf1, sem1).start()
        pltpu.make_async_copy(tab_hbm.at[idx_v.at[c0]], buf0, sem0).wait()
        write_out(buf0, b0 + CHUNK_B * c0)

        @pl.when(c0 + 2 < _CHUNKS_W)
        def _():
            pltpu.make_async_copy(tab_hbm.at[idx_v.at[c0 + 2]], buf0, sem0).start()

        pltpu.make_async_copy(tab_hbm.at[idx_v.at[c0 + 1]], buf1, sem1).wait()
        write_out(buf1, b0 + CHUNK_B * (c0 + 1))
        return carry

    lax.fori_loop(0, _CHUNKS_W // 2, body, 0)


def kernel(x, E_object_index):
    x2 = x.astype(jnp.int32).reshape(BATCH // CHUNK_B, CHUNK_I)
    xp = jnp.concatenate([x2, x2[:, CHUNK_I - (CHUNK_IP - CHUNK_I):]], axis=1)
    xp = xp.reshape(SPLITS, _NW, _CHUNKS_W, CHUNK_IP)
    outs = [_gather_kernel(E_object_index, xp[s]) for s in range(SPLITS)]
    return jnp.concatenate(outs, axis=0)


# speedup vs baseline: 1.0702x; 1.0702x over previous
import functools

import jax
import jax.numpy as jnp
from jax import lax
from jax.experimental import pallas as pl
from jax.experimental.pallas import tpu as pltpu
from jax.experimental.pallas import tpu_sc as plsc

BATCH = 4096
HIST = 50
E_DIMS = 64
CHUNK_B = 2
CHUNK_I = CHUNK_B * HIST
CHUNK_IP = 104
NBUF = 4

_info = plsc.get_sparse_core_info()
_NC, _NS = _info.num_cores, _info.num_subcores
_NW = _NC * _NS
_ROWS_W = BATCH // _NW
_CHUNKS_W = _ROWS_W // CHUNK_B

_mesh = plsc.VectorSubcoreMesh(core_axis_name="c", subcore_axis_name="s")


@functools.partial(
    pl.kernel,
    mesh=_mesh,
    out_type=jax.ShapeDtypeStruct((BATCH, HIST, E_DIMS), jnp.float32),
    scratch_types=[
        pltpu.VMEM((_CHUNKS_W, CHUNK_IP), jnp.int32),
        [pltpu.VMEM((CHUNK_IP, E_DIMS), jnp.float32) for _ in range(NBUF)],
        [pltpu.SemaphoreType.DMA for _ in range(NBUF)],
        [pltpu.SemaphoreType.DMA for _ in range(NBUF)],
    ],
    compiler_params=pltpu.CompilerParams(use_tc_tiling_on_sc=False),
)
def _gather_kernel(tab_hbm, xp_hbm, out_hbm, idx_v, bufs, gsems, wsems):
    wid = lax.axis_index("s") * _NC + lax.axis_index("c")
    b0 = wid * _ROWS_W
    pltpu.sync_copy(xp_hbm.at[wid], idx_v)

    def start_gather(c, k):
        pltpu.make_async_copy(tab_hbm.at[idx_v.at[c]], bufs[k], gsems[k]).start()

    def wait_gather(c, k):
        pltpu.make_async_copy(tab_hbm.at[idx_v.at[c]], bufs[k], gsems[k]).wait()

    def start_write(c, k):
        b = b0 + CHUNK_B * c
        pltpu.make_async_copy(
            bufs[k].at[pl.ds(0, HIST), :], out_hbm.at[b], wsems[k]
        ).start()
        pltpu.make_async_copy(
            bufs[k].at[pl.ds(HIST, HIST), :], out_hbm.at[b + 1], wsems[k]
        ).start()

    def wait_write(c, k):
        b = b0 + CHUNK_B * c
        pltpu.make_async_copy(
            bufs[k].at[pl.ds(0, HIST), :], out_hbm.at[b], wsems[k]
        ).wait()
        pltpu.make_async_copy(
            bufs[k].at[pl.ds(HIST, HIST), :], out_hbm.at[b + 1], wsems[k]
        ).wait()

    start_gather(0, 0)
    start_gather(1, 1)

    def body(i, carry):
        for k in range(NBUF):
            c = NBUF * i + k
            k2 = (k + 2) % NBUF
            wait_gather(c, k)
            start_write(c, k)

            @pl.when(c + 2 < _CHUNKS_W)
            def _():
                @pl.when(c >= 2)
                def _():
                    wait_write(c - 2, k2)

                start_gather(c + 2, k2)

        return carry

    lax.fori_loop(0, _CHUNKS_W // NBUF, body, 0)
    wait_write(_CHUNKS_W - 2, (_CHUNKS_W - 2) % NBUF)
    wait_write(_CHUNKS_W - 1, (_CHUNKS_W - 1) % NBUF)


def kernel(x, E_object_index):
    x2 = x.astype(jnp.int32).reshape(BATCH // CHUNK_B, CHUNK_I)
    xp = jnp.concatenate([x2, x2[:, CHUNK_I - (CHUNK_IP - CHUNK_I):]], axis=1)
    xp = xp.reshape(_NW, _CHUNKS_W, CHUNK_IP)
    return _gather_kernel(E_object_index, xp)
